# baseline (speedup 1.0000x reference)
import functools

import jax
import jax.numpy as jnp
from jax import lax
from jax.experimental import pallas as pl
from jax.experimental.pallas import tpu as pltpu
from jax.experimental.pallas import tpu_sc as plsc

K = 8192
CODE_DIM = 64
D_IN = 256
D_HID = 512

BR = 1024
KB = 4096
BR_DEC = 2048


def _bf16(x):
    return x.astype(jnp.bfloat16).astype(jnp.float32)


def _enc_vq_body(x_ref, w1_ref, b1_ref, w2_ref, b2_ref, cb_ref, h_ref, idx_ref):
    x = x_ref[...]
    h1 = jnp.dot(x, w1_ref[...]) + b1_ref[...]
    h1 = _bf16(jnp.maximum(h1, 0.0))
    h = jnp.dot(h1, w2_ref[...]) + b2_ref[...]
    h_ref[...] = h

    h2 = jnp.sum(h * h, axis=1, keepdims=True)
    hb = _bf16(h)

    run_m = jnp.full((BR, 1), jnp.inf, jnp.float32)
    run_i = jnp.zeros((BR,), jnp.int32)
    for c in range(K // KB):
        cb_c = cb_ref[pl.ds(c * KB, KB), :]
        c2 = jnp.sum(cb_c * cb_c, axis=1)
        dot = lax.dot_general(hb, cb_c, (((1,), (1,)), ((), ())))
        d = (h2 - 2.0 * dot) + c2[None, :]
        m = jnp.min(d, axis=1, keepdims=True)
        li = jnp.argmin(d, axis=1).astype(jnp.int32) + c * KB
        better = m < run_m
        run_i = jnp.where(better[:, 0], li, run_i)
        run_m = _bf16(jnp.where(better, m, run_m))
    idx_ref[0, 0, :] = run_i


def _enc_vq(x2d, w1, b1, w2, b2, cb):
    n = x2d.shape[0]
    nb = n // BR
    h, idx3 = pl.pallas_call(
        _enc_vq_body,
        grid=(nb,),
        in_specs=[
            pl.BlockSpec((BR, D_IN), lambda i: (i, 0)),
            pl.BlockSpec((D_IN, D_HID), lambda i: (0, 0)),
            pl.BlockSpec((1, D_HID), lambda i: (0, 0)),
            pl.BlockSpec((D_HID, CODE_DIM), lambda i: (0, 0)),
            pl.BlockSpec((1, CODE_DIM), lambda i: (0, 0)),
            pl.BlockSpec((K, CODE_DIM), lambda i: (0, 0)),
        ],
        out_specs=[
            pl.BlockSpec((BR, CODE_DIM), lambda i: (i, 0)),
            pl.BlockSpec((1, 1, BR), lambda i: (i, 0, 0)),
        ],
        out_shape=[
            jax.ShapeDtypeStruct((n, CODE_DIM), jnp.float32),
            jax.ShapeDtypeStruct((nb, 1, BR), jnp.int32),
        ],
    )(x2d, w1, b1.reshape(1, D_HID), w2, b2.reshape(1, CODE_DIM), cb)
    return h, idx3.reshape(n)


GATHER_W = 128


def _make_sc_gather(n):
    info = plsc.get_sparse_core_info()
    nw = info.num_cores * info.num_subcores
    bpw = n // nw
    mesh = plsc.VectorSubcoreMesh(core_axis_name="c", subcore_axis_name="s")

    @functools.partial(
        pl.kernel,
        mesh=mesh,
        out_type=jax.ShapeDtypeStruct((n, GATHER_W), jnp.float32),
        scratch_types=[
            pltpu.VMEM((bpw,), jnp.int32),
            pltpu.VMEM((bpw, GATHER_W), jnp.float32),
            pltpu.SemaphoreType.DMA,
        ],
    )
    def gather_k(idx_hbm, table_hbm, out_hbm, idx_v, rows_v, sem):
        wid = lax.axis_index("s") * info.num_cores + lax.axis_index("c")
        base = wid * bpw
        pltpu.sync_copy(idx_hbm.at[pl.ds(base, bpw)], idx_v)
        pltpu.async_copy(table_hbm.at[idx_v], rows_v, sem).wait()
        pltpu.sync_copy(rows_v, out_hbm.at[pl.ds(base, bpw)])

    return gather_k


def _dec_body(h_ref, q_ref, w1_ref, b1_ref, w2_ref, b2_ref, z_ref, y_ref):
    h = h_ref[...]
    q = q_ref[:, :CODE_DIM]
    z = h + (q - h)
    z_ref[...] = z
    y1 = jnp.dot(z, w1_ref[...]) + b1_ref[...]
    y1 = _bf16(jnp.maximum(y1, 0.0))
    y_ref[...] = jnp.dot(y1, w2_ref[...]) + b2_ref[...]


def _decode(h, q, w1, b1, w2, b2):
    n = h.shape[0]
    nb = n // BR_DEC
    return pl.pallas_call(
        _dec_body,
        grid=(nb,),
        in_specs=[
            pl.BlockSpec((BR_DEC, CODE_DIM), lambda i: (i, 0)),
            pl.BlockSpec((BR_DEC, GATHER_W), lambda i: (i, 0)),

            pl.BlockSpec((CODE_DIM, D_HID), lambda i: (0, 0)),
            pl.BlockSpec((1, D_HID), lambda i: (0, 0)),
            pl.BlockSpec((D_HID, D_IN), lambda i: (0, 0)),
            pl.BlockSpec((1, D_IN), lambda i: (0, 0)),
        ],
        out_specs=[
            pl.BlockSpec((BR_DEC, CODE_DIM), lambda i: (i, 0)),
            pl.BlockSpec((BR_DEC, D_IN), lambda i: (i, 0)),
        ],
        out_shape=[
            jax.ShapeDtypeStruct((n, CODE_DIM), jnp.float32),
            jax.ShapeDtypeStruct((n, D_IN), jnp.float32),
        ],
    )(h, q, w1, b1.reshape(1, D_HID), w2, b2.reshape(1, D_IN))


def kernel(x, enc_w1, enc_b1, enc_w2, enc_b2, codebook, dec_w1, dec_b1, dec_w2, dec_b2):
    b, t, _ = x.shape
    n = b * t
    nh = n // 2
    x2d = x.reshape(n, D_IN)
    cb_pad = jnp.concatenate(
        [codebook, jnp.zeros((K, GATHER_W - CODE_DIM), jnp.float32)], axis=1)
    gather = _make_sc_gather(nh)
    h0, i0 = _enc_vq(x2d[:nh], enc_w1, enc_b1, enc_w2, enc_b2, codebook)
    q0 = gather(i0, cb_pad)
    h1, i1 = _enc_vq(x2d[nh:], enc_w1, enc_b1, enc_w2, enc_b2, codebook)
    q1 = gather(i1, cb_pad)
    z0, y0 = _decode(h0, q0, dec_w1, dec_b1, dec_w2, dec_b2)
    z1, y1 = _decode(h1, q1, dec_w1, dec_b1, dec_w2, dec_b2)
    z = jnp.concatenate([z0, z1], axis=0)
    y = jnp.concatenate([y0, y1], axis=0)
    return (z.reshape(b, t, CODE_DIM), y.reshape(b, t, D_IN))


# speedup vs baseline: 1.1724x; 1.1724x over previous
import functools

import jax
import jax.numpy as jnp
from jax import lax
from jax.experimental import pallas as pl
from jax.experimental.pallas import tpu as pltpu
from jax.experimental.pallas import tpu_sc as plsc

K = 8192
CODE_DIM = 64
D_IN = 256
D_HID = 512

BR = 2048
KB = 4096
BR_DEC = 2048


def _bf16(x):
    return x.astype(jnp.bfloat16).astype(jnp.float32)


def _enc_vq_body(x_ref, w1_ref, b1_ref, w2_ref, b2_ref, cb_ref, h_ref, idx_ref):
    x = x_ref[...]
    h1 = jnp.dot(x, w1_ref[...]) + b1_ref[...]
    h1 = _bf16(jnp.maximum(h1, 0.0))
    h = jnp.dot(h1, w2_ref[...]) + b2_ref[...]
    h_ref[...] = h

    h2 = jnp.sum(h * h, axis=1, keepdims=True)
    hb = _bf16(h)

    run_m = jnp.full((BR, 1), jnp.inf, jnp.float32)
    run_i = jnp.zeros((BR,), jnp.int32)
    for c in range(K // KB):
        cb_c = cb_ref[pl.ds(c * KB, KB), :]
        c2 = jnp.sum(cb_c * cb_c, axis=1)
        dot = lax.dot_general(hb, cb_c, (((1,), (1,)), ((), ())))
        d = (h2 - 2.0 * dot) + c2[None, :]
        m = jnp.min(d, axis=1, keepdims=True)
        li = jnp.argmin(d, axis=1).astype(jnp.int32) + c * KB
        better = m < run_m
        run_i = jnp.where(better[:, 0], li, run_i)
        run_m = _bf16(jnp.where(better, m, run_m))
    idx_ref[0, 0, :] = run_i


def _enc_vq(x2d, w1, b1, w2, b2, cb):
    n = x2d.shape[0]
    nb = n // BR
    h, idx3 = pl.pallas_call(
        _enc_vq_body,
        grid=(nb,),
        in_specs=[
            pl.BlockSpec((BR, D_IN), lambda i: (i, 0)),
            pl.BlockSpec((D_IN, D_HID), lambda i: (0, 0)),
            pl.BlockSpec((1, D_HID), lambda i: (0, 0)),
            pl.BlockSpec((D_HID, CODE_DIM), lambda i: (0, 0)),
            pl.BlockSpec((1, CODE_DIM), lambda i: (0, 0)),
            pl.BlockSpec((K, CODE_DIM), lambda i: (0, 0)),
        ],
        out_specs=[
            pl.BlockSpec((BR, CODE_DIM), lambda i: (i, 0)),
            pl.BlockSpec((1, 1, BR), lambda i: (i, 0, 0)),
        ],
        out_shape=[
            jax.ShapeDtypeStruct((n, CODE_DIM), jnp.float32),
            jax.ShapeDtypeStruct((nb, 1, BR), jnp.int32),
        ],
    )(x2d, w1, b1.reshape(1, D_HID), w2, b2.reshape(1, CODE_DIM), cb)
    return h, idx3.reshape(n)


GATHER_W = 128


def _make_sc_gather(n):
    info = plsc.get_sparse_core_info()
    nw = info.num_cores * info.num_subcores
    bpw = n // nw
    mesh = plsc.VectorSubcoreMesh(core_axis_name="c", subcore_axis_name="s")

    @functools.partial(
        pl.kernel,
        mesh=mesh,
        out_type=jax.ShapeDtypeStruct((n, GATHER_W), jnp.float32),
        scratch_types=[
            pltpu.VMEM((bpw,), jnp.int32),
            pltpu.VMEM((bpw, GATHER_W), jnp.float32),
            pltpu.SemaphoreType.DMA,
        ],
    )
    def gather_k(idx_hbm, table_hbm, out_hbm, idx_v, rows_v, sem):
        wid = lax.axis_index("s") * info.num_cores + lax.axis_index("c")
        base = wid * bpw
        pltpu.sync_copy(idx_hbm.at[pl.ds(base, bpw)], idx_v)
        pltpu.async_copy(table_hbm.at[idx_v], rows_v, sem).wait()
        pltpu.sync_copy(rows_v, out_hbm.at[pl.ds(base, bpw)])

    return gather_k


def _dec_body(h_ref, q_ref, w1_ref, b1_ref, w2_ref, b2_ref, z_ref, y_ref):
    h = h_ref[...]
    q = q_ref[:, :CODE_DIM]
    z = h + (q - h)
    z_ref[...] = z
    y1 = jnp.dot(z, w1_ref[...]) + b1_ref[...]
    y1 = _bf16(jnp.maximum(y1, 0.0))
    y_ref[...] = jnp.dot(y1, w2_ref[...]) + b2_ref[...]


def _decode(h, q, w1, b1, w2, b2):
    n = h.shape[0]
    nb = n // BR_DEC
    return pl.pallas_call(
        _dec_body,
        grid=(nb,),
        in_specs=[
            pl.BlockSpec((BR_DEC, CODE_DIM), lambda i: (i, 0)),
            pl.BlockSpec((BR_DEC, GATHER_W), lambda i: (i, 0)),

            pl.BlockSpec((CODE_DIM, D_HID), lambda i: (0, 0)),
            pl.BlockSpec((1, D_HID), lambda i: (0, 0)),
            pl.BlockSpec((D_HID, D_IN), lambda i: (0, 0)),
            pl.BlockSpec((1, D_IN), lambda i: (0, 0)),
        ],
        out_specs=[
            pl.BlockSpec((BR_DEC, CODE_DIM), lambda i: (i, 0)),
            pl.BlockSpec((BR_DEC, D_IN), lambda i: (i, 0)),
        ],
        out_shape=[
            jax.ShapeDtypeStruct((n, CODE_DIM), jnp.float32),
            jax.ShapeDtypeStruct((n, D_IN), jnp.float32),
        ],
    )(h, q, w1, b1.reshape(1, D_HID), w2, b2.reshape(1, D_IN))


def kernel(x, enc_w1, enc_b1, enc_w2, enc_b2, codebook, dec_w1, dec_b1, dec_w2, dec_b2):
    b, t, _ = x.shape
    n = b * t
    x2d = x.reshape(n, D_IN)
    h, idx = _enc_vq(x2d, enc_w1, enc_b1, enc_w2, enc_b2, codebook)
    cb_pad = jnp.concatenate(
        [codebook, jnp.zeros((K, GATHER_W - CODE_DIM), jnp.float32)], axis=1)
    q = _make_sc_gather(n)(idx, cb_pad)
    z, y = _decode(h, q, dec_w1, dec_b1, dec_w2, dec_b2)
    return (z.reshape(b, t, CODE_DIM), y.reshape(b, t, D_IN))


# speedup vs baseline: 1.2182x; 1.0391x over previous
import functools

import jax
import jax.numpy as jnp
from jax import lax
from jax.experimental import pallas as pl
from jax.experimental.pallas import tpu as pltpu
from jax.experimental.pallas import tpu_sc as plsc

K = 8192
CODE_DIM = 64
D_IN = 256
D_HID = 512

BR = 2048
KB = 4096
BR_DEC = 8192


def _bf16(x):
    return x.astype(jnp.bfloat16).astype(jnp.float32)


def _enc_vq_body(x_ref, w1_ref, b1_ref, w2_ref, b2_ref, cb_ref, h_ref, idx_ref):
    x = x_ref[...]
    h1 = jnp.dot(x, w1_ref[...]) + b1_ref[...]
    h1 = _bf16(jnp.maximum(h1, 0.0))
    h = jnp.dot(h1, w2_ref[...]) + b2_ref[...]
    h_ref[...] = h

    h2 = jnp.sum(h * h, axis=1, keepdims=True)
    hb2 = _bf16(h) * 2.0

    run_m = jnp.full((BR, 1), jnp.inf, jnp.float32)
    run_i = jnp.zeros((BR,), jnp.int32)
    for c in range(K // KB):
        cb_c = cb_ref[pl.ds(c * KB, KB), :]
        c2 = jnp.sum(cb_c * cb_c, axis=1)
        dot2 = lax.dot_general(hb2, cb_c, (((1,), (1,)), ((), ())))
        d = (h2 - dot2) + c2[None, :]
        m = jnp.min(d, axis=1, keepdims=True)
        li = jnp.argmin(d, axis=1).astype(jnp.int32) + c * KB
        better = m < run_m
        run_i = jnp.where(better[:, 0], li, run_i)
        run_m = _bf16(jnp.where(better, m, run_m))
    idx_ref[0, 0, :] = run_i


def _enc_vq(x2d, w1, b1, w2, b2, cb):
    n = x2d.shape[0]
    nb = n // BR
    h, idx3 = pl.pallas_call(
        _enc_vq_body,
        grid=(nb,),
        in_specs=[
            pl.BlockSpec((BR, D_IN), lambda i: (i, 0)),
            pl.BlockSpec((D_IN, D_HID), lambda i: (0, 0)),
            pl.BlockSpec((1, D_HID), lambda i: (0, 0)),
            pl.BlockSpec((D_HID, CODE_DIM), lambda i: (0, 0)),
            pl.BlockSpec((1, CODE_DIM), lambda i: (0, 0)),
            pl.BlockSpec((K, CODE_DIM), lambda i: (0, 0)),
        ],
        out_specs=[
            pl.BlockSpec((BR, CODE_DIM), lambda i: (i, 0)),
            pl.BlockSpec((1, 1, BR), lambda i: (i, 0, 0)),
        ],
        out_shape=[
            jax.ShapeDtypeStruct((n, CODE_DIM), jnp.float32),
            jax.ShapeDtypeStruct((nb, 1, BR), jnp.int32),
        ],
    )(x2d, w1, b1.reshape(1, D_HID), w2, b2.reshape(1, CODE_DIM), cb)
    return h, idx3.reshape(n)


GATHER_W = 128


def _make_sc_gather(n):
    info = plsc.get_sparse_core_info()
    nw = info.num_cores * info.num_subcores
    bpw = n // nw
    mesh = plsc.VectorSubcoreMesh(core_axis_name="c", subcore_axis_name="s")

    @functools.partial(
        pl.kernel,
        mesh=mesh,
        out_type=jax.ShapeDtypeStruct((n, GATHER_W), jnp.float32),
        scratch_types=[
            pltpu.VMEM((bpw,), jnp.int32),
            pltpu.VMEM((bpw, GATHER_W), jnp.float32),
            pltpu.SemaphoreType.DMA,
        ],
    )
    def gather_k(idx_hbm, table_hbm, out_hbm, idx_v, rows_v, sem):
        wid = lax.axis_index("s") * info.num_cores + lax.axis_index("c")
        base = wid * bpw
        pltpu.sync_copy(idx_hbm.at[pl.ds(base, bpw)], idx_v)
        pltpu.async_copy(table_hbm.at[idx_v], rows_v, sem).wait()
        pltpu.sync_copy(rows_v, out_hbm.at[pl.ds(base, bpw)])

    return gather_k


def _dec_body(h_ref, q_ref, w1_ref, b1_ref, w2_ref, b2_ref, z_ref, y_ref):
    h = h_ref[...]
    q = q_ref[:, :CODE_DIM]
    z = h + (q - h)
    z_ref[...] = z
    y1 = jnp.dot(z, w1_ref[...]) + b1_ref[...]
    y1 = _bf16(jnp.maximum(y1, 0.0))
    y_ref[...] = jnp.dot(y1, w2_ref[...]) + b2_ref[...]


def _decode(h, q, w1, b1, w2, b2):
    n = h.shape[0]
    nb = n // BR_DEC
    return pl.pallas_call(
        _dec_body,
        grid=(nb,),
        in_specs=[
            pl.BlockSpec((BR_DEC, CODE_DIM), lambda i: (i, 0)),
            pl.BlockSpec((BR_DEC, GATHER_W), lambda i: (i, 0)),

            pl.BlockSpec((CODE_DIM, D_HID), lambda i: (0, 0)),
            pl.BlockSpec((1, D_HID), lambda i: (0, 0)),
            pl.BlockSpec((D_HID, D_IN), lambda i: (0, 0)),
            pl.BlockSpec((1, D_IN), lambda i: (0, 0)),
        ],
        out_specs=[
            pl.BlockSpec((BR_DEC, CODE_DIM), lambda i: (i, 0)),
            pl.BlockSpec((BR_DEC, D_IN), lambda i: (i, 0)),
        ],
        out_shape=[
            jax.ShapeDtypeStruct((n, CODE_DIM), jnp.float32),
            jax.ShapeDtypeStruct((n, D_IN), jnp.float32),
        ],
    )(h, q, w1, b1.reshape(1, D_HID), w2, b2.reshape(1, D_IN))


def kernel(x, enc_w1, enc_b1, enc_w2, enc_b2, codebook, dec_w1, dec_b1, dec_w2, dec_b2):
    b, t, _ = x.shape
    n = b * t
    x2d = x.reshape(n, D_IN)
    h, idx = _enc_vq(x2d, enc_w1, enc_b1, enc_w2, enc_b2, codebook)
    cb_pad = jnp.concatenate(
        [codebook, jnp.zeros((K, GATHER_W - CODE_DIM), jnp.float32)], axis=1)
    q = _make_sc_gather(n)(idx, cb_pad)
    z, y = _decode(h, q, dec_w1, dec_b1, dec_w2, dec_b2)
    return (z.reshape(b, t, CODE_DIM), y.reshape(b, t, D_IN))
